# baseline (speedup 1.0000x reference)
import functools

import jax
import jax.numpy as jnp
from jax import lax
from jax.experimental import pallas as pl
from jax.experimental.pallas import tpu as pltpu
from jax.experimental.pallas import tpu_sc as plsc

NC = 2
NS = 16
NW = NC * NS
L = 16

W = 512
CHUNK_ROWS = 16
NBUF = 3


def _sc_lut_kernel(n_rows: int):
    rows_per_w = n_rows // NW
    n_chunks = rows_per_w // CHUNK_ROWS
    assert n_chunks % NBUF == 0 and n_chunks >= 2 * NBUF
    mesh = plsc.VectorSubcoreMesh(core_axis_name="c", subcore_axis_name="s")

    buf_t = pltpu.VMEM((CHUNK_ROWS, W), jnp.float32)

    @functools.partial(
        pl.kernel,
        out_type=jax.ShapeDtypeStruct((3 * n_rows, W), jnp.float32),
        mesh=mesh,
        compiler_params=pltpu.CompilerParams(
            needs_layout_passes=False, use_tc_tiling_on_sc=True
        ),
        scratch_types=(
            [pltpu.VMEM((8, 128), jnp.float32)]
            + [buf_t] * NBUF
            + [buf_t] * (3 * NBUF)
            + [pltpu.SemaphoreType.DMA] * (2 * NBUF)
        ),
    )
    def body(in_hbm, lut_hbm, out_hbm, lut_v, *rest):
        cid = lax.axis_index("c")
        sid = lax.axis_index("s")
        wid = sid * NC + cid
        pltpu.sync_copy(lut_hbm, lut_v)
        base = wid * rows_per_w

        in_bufs = rest[:NBUF]
        out_bufs = tuple(
            rest[NBUF + 3 * s: NBUF + 3 * s + 3] for s in range(NBUF)
        )
        in_sems = rest[4 * NBUF: 5 * NBUF]
        out_sems = rest[5 * NBUF: 6 * NBUF]
        zero16 = jnp.zeros((L,), jnp.int32)
        scale = jnp.float32(32.0 / 255.0)

        def in_slice(ci):
            r0 = pl.multiple_of(base + ci * CHUNK_ROWS, CHUNK_ROWS)
            return in_hbm.at[pl.ds(r0, CHUNK_ROWS)]

        def out_slice(ci, j):
            r0 = base + ci * CHUNK_ROWS
            h0 = r0 & 511
            ob = pl.multiple_of(3 * (r0 - h0) + h0 + 512 * j, CHUNK_ROWS)
            return out_hbm.at[pl.ds(ob, CHUNK_ROWS)]

        def start_in(ci, s):
            pltpu.async_copy(in_slice(ci), in_bufs[s], in_sems[s])

        def wait_in(s):
            pltpu.make_async_copy(in_slice(0), in_bufs[s], in_sems[s]).wait()

        def start_out(ci, s):
            for j in range(3):
                pltpu.async_copy(out_bufs[s][j], out_slice(ci, j), out_sems[s])

        def wait_out(s):
            for j in range(3):
                pltpu.make_async_copy(
                    out_bufs[s][j], out_slice(0, j), out_sems[s]
                ).wait()

        def compute(s):
            in_v = in_bufs[s]
            outs = out_bufs[s]

            def row_body(r, c2):
                for g in range(0, W // L, 8):
                    ks = range(8)
                    xs = [in_v[r, pl.ds((g + k) * L, L)] for k in ks]
                    idxs = [
                        jnp.clip((x * scale).astype(jnp.int32), 0, 32)
                        for x in xs
                    ]
                    vals = [
                        [
                            plsc.load_gather(lut_v, [zero16, i + 33 * j])
                            for i in idxs
                        ]
                        for j in range(3)
                    ]
                    for j in range(3):
                        for k in ks:
                            outs[j][r, pl.ds((g + k) * L, L)] = vals[j][k]
                return c2

            lax.fori_loop(0, CHUNK_ROWS, row_body, 0)

        for s in range(NBUF):
            start_in(s, s)
        for ci in range(NBUF):
            wait_in(ci)
            compute(ci)
            start_in(ci + NBUF, ci)
            start_out(ci, ci)

        def super_body(ii, carry):
            for s in range(NBUF):
                ci = ii * NBUF + s
                wait_in(s)
                wait_out(s)
                compute(s)
                start_in(ci + NBUF, s)
                start_out(ci, s)
            return carry

        lax.fori_loop(1, n_chunks // NBUF - 1, super_body, 0)

        for s in range(NBUF):
            ci = n_chunks - NBUF + s
            wait_in(s)
            wait_out(s)
            compute(s)
            start_out(ci, s)
        for s in range(NBUF):
            wait_out(s)

    return body


def kernel(input, lut):
    b, h, w, c = input.shape
    n_rows = b * c * h
    x2d = jnp.transpose(input, (0, 3, 1, 2)).reshape((n_rows, w))
    lut_t = jnp.pad(lut.T.reshape((lut.size,)), (0, 1024 - lut.size))
    out = _sc_lut_kernel(n_rows)(x2d, lut_t.reshape((8, 128)))
    z = out.reshape((b, c, 3, h, w))
    return jnp.transpose(z, (0, 3, 4, 1, 2))


# speedup vs baseline: 1.0297x; 1.0297x over previous
import functools

import jax
import jax.numpy as jnp
from jax import lax
from jax.experimental import pallas as pl
from jax.experimental.pallas import tpu as pltpu
from jax.experimental.pallas import tpu_sc as plsc

NC = 2
NS = 16
NW = NC * NS
L = 16

W = 512
CHUNK_ROWS = 16
NBUF = 2


def _sc_lut_kernel(n_rows: int):
    rows_per_w = n_rows // NW
    n_chunks = rows_per_w // CHUNK_ROWS
    assert n_chunks % NBUF == 0 and n_chunks >= 2 * NBUF
    mesh = plsc.VectorSubcoreMesh(core_axis_name="c", subcore_axis_name="s")

    buf_t = pltpu.VMEM((CHUNK_ROWS, W), jnp.float32)

    @functools.partial(
        pl.kernel,
        out_type=jax.ShapeDtypeStruct((3 * n_rows, W), jnp.float32),
        mesh=mesh,
        compiler_params=pltpu.CompilerParams(
            needs_layout_passes=False, use_tc_tiling_on_sc=True
        ),
        scratch_types=[
            pltpu.VMEM((8, 128), jnp.float32),
            buf_t, buf_t,
            buf_t, buf_t, buf_t,
            buf_t, buf_t, buf_t,
            pltpu.SemaphoreType.DMA,
            pltpu.SemaphoreType.DMA,
            pltpu.SemaphoreType.DMA,
            pltpu.SemaphoreType.DMA,
        ],
    )
    def body(in_hbm, lut_hbm, out_hbm, lut_v,
             i0, i1, a0, a1, a2, b0, b1, b2,
             si0, si1, so0, so1):
        cid = lax.axis_index("c")
        sid = lax.axis_index("s")
        wid = sid * NC + cid
        pltpu.sync_copy(lut_hbm, lut_v)
        base = wid * rows_per_w

        in_bufs = (i0, i1)
        out_bufs = ((a0, a1, a2), (b0, b1, b2))
        in_sems = (si0, si1)
        out_sems = (so0, so1)
        zero16 = jnp.zeros((L,), jnp.int32)
        scale = jnp.float32(32.0 / 255.0)

        def in_slice(ci):
            r0 = pl.multiple_of(base + ci * CHUNK_ROWS, CHUNK_ROWS)
            return in_hbm.at[pl.ds(r0, CHUNK_ROWS)]

        def out_slice(ci, j):
            r0 = base + ci * CHUNK_ROWS
            h0 = r0 & 511
            ob = pl.multiple_of(3 * (r0 - h0) + h0 + 512 * j, CHUNK_ROWS)
            return out_hbm.at[pl.ds(ob, CHUNK_ROWS)]

        def start_in(ci, s):
            pltpu.async_copy(in_slice(ci), in_bufs[s], in_sems[s])

        def wait_in(s):
            pltpu.make_async_copy(in_slice(0), in_bufs[s], in_sems[s]).wait()

        def start_out(ci, s):
            for j in range(3):
                pltpu.async_copy(out_bufs[s][j], out_slice(ci, j), out_sems[s])

        def wait_out(s):
            for j in range(3):
                pltpu.make_async_copy(
                    out_bufs[s][j], out_slice(0, j), out_sems[s]
                ).wait()

        def compute(s):
            in_v = in_bufs[s]
            outs = out_bufs[s]

            def row_body(r, c2):
                for g in range(0, W // L, 8):
                    ks = range(8)
                    xs = [in_v[r, pl.ds((g + k) * L, L)] for k in ks]
                    idxs = [
                        jnp.clip((x * scale).astype(jnp.int32), 0, 32)
                        for x in xs
                    ]
                    vals = [
                        [
                            plsc.load_gather(lut_v, [zero16, i + 33 * j])
                            for i in idxs
                        ]
                        for j in range(3)
                    ]
                    for j in range(3):
                        for k in ks:
                            outs[j][r, pl.ds((g + k) * L, L)] = vals[j][k]
                return c2

            lax.fori_loop(0, CHUNK_ROWS, row_body, 0)

        start_in(0, 0)
        start_in(1, 1)
        for ci in range(2):
            wait_in(ci)
            compute(ci)
            start_in(ci + 2, ci)
            start_out(ci, ci)

        def super_body(ii, carry):
            for s in range(NBUF):
                ci = ii * NBUF + s
                wait_in(s)
                wait_out(s)
                compute(s)
                start_in(ci + 2, s)
                start_out(ci, s)
            return carry

        lax.fori_loop(1, n_chunks // NBUF - 1, super_body, 0)

        for s in range(2):
            ci = n_chunks - 2 + s
            wait_in(s)
            wait_out(s)
            compute(s)
            start_out(ci, s)
        for s in range(2):
            wait_out(s)

    return body


def kernel(input, lut):
    b, h, w, c = input.shape
    n_rows = b * c * h
    x2d = jnp.transpose(input, (0, 3, 1, 2)).reshape((n_rows, w))
    lut_t = jnp.pad(lut.T.reshape((lut.size,)), (0, 1024 - lut.size))
    out = _sc_lut_kernel(n_rows)(x2d, lut_t.reshape((8, 128)))
    z = out.reshape((b, c, 3, h, w))
    return jnp.transpose(z, (0, 3, 4, 1, 2))
